# Initial kernel scaffold; baseline (speedup 1.0000x reference)
import jax
import jax.numpy as jnp
from jax.experimental import pallas as pl


def kernel(seq_len_q, seq_len_k, embedding):
    raise NotImplementedError("write your pallas kernel here")


# speedup vs baseline: 42.5523x; 42.5523x over previous
import functools

import jax
import jax.numpy as jnp
from jax import lax
from jax.experimental import pallas as pl
from jax.experimental.pallas import tpu as pltpu
from jax.experimental.pallas import tpu_sc as plsc

NUM_HEADS = 16
NUM_BUCKETS = 32
SEQ = 2048
W = 4096
DPAD = 4224
NSHIFT = 8


def _table_body(delta_ref, embt_ref, out_ref):
    delta = delta_ref[0, 0]
    d = lax.broadcasted_iota(jnp.int32, (1, DPAD), 1)
    rel = d - (SEQ - 1) + delta
    n = -rel
    side = jnp.where(n < 0, 16, 0).astype(jnp.int32)
    na = jnp.abs(n)
    large = jnp.full(na.shape, 8, jnp.int32)
    for j in range(1, 16):
        large = large + (na >= (1 << (j + 3))).astype(jnp.int32)
    bucket = side + jnp.where(na < 8, na, large)
    bucket = jnp.minimum(bucket, NUM_BUCKETS - 1)
    rows = lax.broadcasted_iota(jnp.int32, (NUM_BUCKETS, DPAD), 0)
    oh = (jnp.broadcast_to(bucket, (NUM_BUCKETS, DPAD)) == rows).astype(jnp.float32)
    vflat = lax.dot_general(embt_ref[...], oh, (((1,), (0,)), ((), ())),
                            precision=lax.Precision.HIGHEST,
                            preferred_element_type=jnp.float32)
    for s in range(NSHIFT):
        out_ref[:, s, :] = vflat[:, s:s + W]


def _build_table(embt, delta):
    return pl.pallas_call(
        _table_body,
        out_shape=jax.ShapeDtypeStruct((NUM_HEADS, NSHIFT, W), jnp.float32),
        in_specs=[
            pl.BlockSpec(memory_space=pltpu.SMEM),
            pl.BlockSpec(memory_space=pltpu.VMEM),
        ],
    )(delta, embt)


@functools.cache
def _make_sc_expand():
    mesh = plsc.VectorSubcoreMesh(core_axis_name="c", subcore_axis_name="s")

    @functools.partial(
        pl.kernel,
        mesh=mesh,
        out_type=jax.ShapeDtypeStruct((NUM_HEADS * SEQ * SEQ,), jnp.float32),
        scratch_types=[
            pltpu.VMEM((NSHIFT * W,), jnp.float32),
            pltpu.SemaphoreType.DMA,
        ],
    )
    def _sc_expand(vsh_hbm, out_hbm, vbuf, sem):
        h = lax.axis_index("s")
        half = lax.axis_index("c")
        base = half * (SEQ // 2)
        rbase = h * SEQ + base
        tbl_off = pl.multiple_of(h * (NSHIFT * W), 8)
        pltpu.sync_copy(vsh_hbm.at[pl.ds(tbl_off, NSHIFT * W)], vbuf)

        def body(j, carry):
            ob = 2040 - base - 8 * j
            copies = []
            for t in range(8):
                src_off = pl.multiple_of((7 - t) * W + ob, 8)
                src = vbuf.at[pl.ds(src_off, SEQ)]
                dst_off = pl.multiple_of((rbase + 8 * j + t) * SEQ, 8)
                dst = out_hbm.at[pl.ds(dst_off, SEQ)]
                copies.append(pltpu.async_copy(src, dst, sem))
            for c in copies:
                c.wait()
            return carry

        lax.fori_loop(0, SEQ // 2 // 8, body, 0)

    return _sc_expand


def kernel(seq_len_q, seq_len_k, embedding):
    delta = (jnp.asarray(seq_len_k, jnp.int32)
             - jnp.asarray(seq_len_q, jnp.int32)).reshape(1, 1)
    embt = jnp.transpose(embedding).astype(jnp.float32)
    vsh = _build_table(embt, delta)
    vsh = vsh.reshape(NUM_HEADS * NSHIFT * W)
    out = _make_sc_expand()(vsh)
    return out.reshape(1, NUM_HEADS, SEQ, SEQ)


# speedup vs baseline: 111.9903x; 2.6318x over previous
import functools

import jax
import jax.numpy as jnp
from jax import lax
from jax.experimental import pallas as pl
from jax.experimental.pallas import tpu as pltpu
from jax.experimental.pallas import tpu_sc as plsc

NUM_HEADS = 16
NUM_BUCKETS = 32
SEQ = 2048
WS = 3968
DPAD = 4224
NSHIFT = 16


def _table_body(delta_ref, embt_ref, out_ref, vflat_ref):
    m = pl.program_id(0)

    @pl.when(m == 0)
    def _():
        delta = delta_ref[0, 0]
        d = lax.broadcasted_iota(jnp.int32, (1, DPAD), 1)
        rel = d - (SEQ - 1) + delta
        n = -rel
        side = jnp.where(n < 0, 16, 0).astype(jnp.int32)
        na = jnp.abs(n)
        large = jnp.full(na.shape, 8, jnp.int32)
        for j in range(1, 16):
            large = large + (na >= (1 << (j + 3))).astype(jnp.int32)
        bucket = side + jnp.where(na < 8, na, large)
        bucket = jnp.minimum(bucket, NUM_BUCKETS - 1)
        rows = lax.broadcasted_iota(jnp.int32, (NUM_BUCKETS, DPAD), 0)
        oh = (jnp.broadcast_to(bucket, (NUM_BUCKETS, DPAD)) == rows
              ).astype(jnp.float32)
        vflat_ref[...] = lax.dot_general(
            embt_ref[...], oh, (((1,), (0,)), ((), ())),
            precision=lax.Precision.HIGHEST,
            preferred_element_type=jnp.float32)

    vf = vflat_ref[...]
    for r in range(8):
        t = 8 * m + (7 - r)
        rolled = pltpu.roll(vf, DPAD - t, 1)
        out_ref[:, 0, r, :] = rolled[:, :WS]


def _build_table(embt, delta):
    return pl.pallas_call(
        _table_body,
        grid=(NSHIFT,),
        out_shape=jax.ShapeDtypeStruct((NUM_HEADS, NSHIFT, 8, WS), jnp.float32),
        in_specs=[
            pl.BlockSpec(memory_space=pltpu.SMEM),
            pl.BlockSpec(memory_space=pltpu.VMEM),
        ],
        out_specs=pl.BlockSpec((NUM_HEADS, 1, 8, WS), lambda m: (0, m, 0, 0)),
        scratch_shapes=[pltpu.VMEM((NUM_HEADS, DPAD), jnp.float32)],
    )(delta, embt)


@functools.cache
def _make_sc_expand():
    mesh = plsc.VectorSubcoreMesh(core_axis_name="c", subcore_axis_name="s")

    @functools.partial(
        pl.kernel,
        mesh=mesh,
        out_type=jax.ShapeDtypeStruct((1, NUM_HEADS, SEQ, SEQ), jnp.float32),
        scratch_types=[
            pltpu.VMEM((8, WS), jnp.float32),
            pltpu.VMEM((8, WS), jnp.float32),
            pltpu.VMEM((8, WS), jnp.float32),
            pltpu.VMEM((8, WS), jnp.float32),
            pltpu.SemaphoreType.DMA,
        ],
    )
    def _sc_expand(vshm_hbm, out_hbm, b0, b1, b2, b3, sem):
        h = lax.axis_index("s")
        z = lax.axis_index("c")
        bufs = [b0, b1, b2, b3]
        for pas in range(2):
            for k in range(4):
                mk = 2 * (4 * pas + k) + 1 - z
                pltpu.sync_copy(vshm_hbm.at[h, mk], bufs[k])
            for k in range(4):
                mk = 2 * (4 * pas + k) + 1 - z
                a0 = (255 - mk) & 15
                buf = bufs[k]

                def body(n, carry, buf=buf, a0=a0):
                    a = a0 + 16 * n
                    p0 = (2040 - 8 * a) >> 7
                    qoff = pl.multiple_of(8 * a, 8)
                    copies = []
                    for c in range(16):
                        src = buf.at[:, pl.ds(
                            pl.multiple_of(128 * (p0 + c), 128), 128)]
                        dst = out_hbm.at[0, h, pl.ds(qoff, 8),
                                         pl.ds(128 * c, 128)]
                        copies.append(pltpu.async_copy(src, dst, sem))
                    for cp in copies:
                        cp.wait()
                    return carry

                lax.fori_loop(0, 16, body, 0)

    return _sc_expand


def kernel(seq_len_q, seq_len_k, embedding):
    delta = (jnp.asarray(seq_len_k, jnp.int32)
             - jnp.asarray(seq_len_q, jnp.int32)).reshape(1, 1)
    embt = jnp.transpose(embedding).astype(jnp.float32)
    vshm = _build_table(embt, delta)
    return _make_sc_expand()(vshm)


# speedup vs baseline: 115.3213x; 1.0297x over previous
import functools

import jax
import jax.numpy as jnp
from jax import lax
from jax.experimental import pallas as pl
from jax.experimental.pallas import tpu as pltpu
from jax.experimental.pallas import tpu_sc as plsc

NUM_HEADS = 16
NUM_BUCKETS = 32
SEQ = 2048
WS = 3968
DPAD = 4224
NSHIFT = 16


def _table_body(delta_ref, embt_ref, out_ref, vflat_ref):
    m = pl.program_id(0)

    @pl.when(m == 0)
    def _():
        delta = delta_ref[0, 0]
        d = lax.broadcasted_iota(jnp.int32, (1, DPAD), 1)
        rel = d - (SEQ - 1) + delta
        n = -rel
        side = jnp.where(n < 0, 16, 0).astype(jnp.int32)
        na = jnp.abs(n)
        large = jnp.full(na.shape, 8, jnp.int32)
        for j in range(1, 16):
            large = large + (na >= (1 << (j + 3))).astype(jnp.int32)
        bucket = side + jnp.where(na < 8, na, large)
        bucket = jnp.minimum(bucket, NUM_BUCKETS - 1)
        rows = lax.broadcasted_iota(jnp.int32, (NUM_BUCKETS, DPAD), 0)
        oh = (jnp.broadcast_to(bucket, (NUM_BUCKETS, DPAD)) == rows
              ).astype(jnp.float32)
        vflat_ref[...] = lax.dot_general(
            embt_ref[...], oh, (((1,), (0,)), ((), ())),
            precision=lax.Precision.HIGHEST,
            preferred_element_type=jnp.float32)

    vf = vflat_ref[...]
    for r in range(8):
        t = 8 * m + (7 - r)
        rolled = pltpu.roll(vf, DPAD - t, 1)
        out_ref[:, 0, r, :] = rolled[:, :WS]


def _build_table(embt, delta):
    return pl.pallas_call(
        _table_body,
        grid=(NSHIFT,),
        out_shape=jax.ShapeDtypeStruct((NUM_HEADS, NSHIFT, 8, WS), jnp.float32),
        in_specs=[
            pl.BlockSpec(memory_space=pltpu.SMEM),
            pl.BlockSpec(memory_space=pltpu.VMEM),
        ],
        out_specs=pl.BlockSpec((NUM_HEADS, 1, 8, WS), lambda m: (0, m, 0, 0)),
        scratch_shapes=[pltpu.VMEM((NUM_HEADS, DPAD), jnp.float32)],
    )(delta, embt)


@functools.cache
def _make_sc_expand():
    mesh = plsc.VectorSubcoreMesh(core_axis_name="c", subcore_axis_name="s")

    @functools.partial(
        pl.kernel,
        mesh=mesh,
        out_type=jax.ShapeDtypeStruct((1, NUM_HEADS, SEQ, SEQ), jnp.float32),
        scratch_types=[
            pltpu.VMEM((8, WS), jnp.float32),
            pltpu.VMEM((8, WS), jnp.float32),
            pltpu.VMEM((8, WS), jnp.float32),
            pltpu.VMEM((8, WS), jnp.float32),
            pltpu.SemaphoreType.DMA,
            pltpu.SemaphoreType.DMA,
        ],
    )
    def _sc_expand(vshm_hbm, out_hbm, b0, b1, b2, b3, sem, ldsem):
        h = lax.axis_index("s")
        z = lax.axis_index("c")
        bufs = [b0, b1, b2, b3]
        depth = 6
        pending = []
        for pas in range(2):
            loads = []
            for k in range(4):
                mk = 2 * (4 * pas + k) + 1 - z
                loads.append(pltpu.async_copy(vshm_hbm.at[h, mk], bufs[k], ldsem))
            for ld in loads:
                ld.wait()
            for k in range(4):
                mk = 2 * (4 * pas + k) + 1 - z
                a0 = (255 - mk) & 15
                for n in range(16):
                    a = a0 + 16 * n
                    p0 = (2040 - 8 * a) >> 7
                    src = bufs[k].at[:, pl.ds(
                        pl.multiple_of(128 * p0, 128), SEQ)]
                    dst = out_hbm.at[0, h, pl.ds(pl.multiple_of(8 * a, 8), 8), :]
                    pending.append(pltpu.async_copy(src, dst, sem))
                    if len(pending) > depth:
                        pending.pop(0).wait()
            for cp in pending:
                cp.wait()
            pending = []

    return _sc_expand


def kernel(seq_len_q, seq_len_k, embedding):
    delta = (jnp.asarray(seq_len_k, jnp.int32)
             - jnp.asarray(seq_len_q, jnp.int32)).reshape(1, 1)
    embt = jnp.transpose(embedding).astype(jnp.float32)
    vshm = _build_table(embt, delta)
    return _make_sc_expand()(vshm)
